# Initial kernel scaffold; baseline (speedup 1.0000x reference)
import jax
import jax.numpy as jnp
from jax.experimental import pallas as pl


def kernel(x, router_W, router_b, emb, Wout, bout):
    raise NotImplementedError("write your pallas kernel here")


# speedup vs baseline: 34.2329x; 34.2329x over previous
import functools

import jax
import jax.numpy as jnp
from jax import lax
from jax.experimental import pallas as pl
from jax.experimental.pallas import tpu as pltpu
from jax.experimental.pallas import tpu_sc as plsc

B, S, V, D, E = 32, 512, 512, 128, 8
CH = 64
LANES = 16


def _router_body(x_ref, rw_ref, rb_ref, emb0_ref, woutr_ref, bout_ref,
                 idx0_ref, idx1_ref, w0_ref, w1_ref, t_ref):
    xi = x_ref[...]
    xf = xi.astype(jnp.float32)
    logits = (jnp.dot(xf, rw_ref[...], preferred_element_type=jnp.float32)
              + rb_ref[...])
    iota = lax.broadcasted_iota(jnp.int32, (B, E), 1)
    m0 = jnp.max(logits, axis=1, keepdims=True)
    a0 = jnp.min(jnp.where(logits == m0, iota, E), axis=1)
    masked = jnp.where(iota == a0[:, None], -1e30, logits)
    m1 = jnp.max(masked, axis=1, keepdims=True)
    a1 = jnp.min(jnp.where(masked == m1, iota, E), axis=1)
    d = jnp.exp(m1[:, 0] - m0[:, 0])
    w0 = 1.0 / (1.0 + d)
    idx0_ref[...] = a0[:, None] * V + xi
    idx1_ref[...] = a1[:, None] * V + xi
    w0_ref[...] = jnp.broadcast_to(w0[:, None], (B, 128))
    w1_ref[...] = jnp.broadcast_to((1.0 - w0)[:, None], (B, 128))
    t_ref[...] = (jnp.dot(emb0_ref[...], woutr_ref[...],
                          preferred_element_type=jnp.float32)
                  + jnp.sum(bout_ref[...], axis=0, keepdims=True))


def _router_call(x, router_W, router_b, emb0, woutr, bout):
    return pl.pallas_call(
        _router_body,
        out_shape=(
            jax.ShapeDtypeStruct((B, S), jnp.int32),
            jax.ShapeDtypeStruct((B, S), jnp.int32),
            jax.ShapeDtypeStruct((B, 128), jnp.float32),
            jax.ShapeDtypeStruct((B, 128), jnp.float32),
            jax.ShapeDtypeStruct((1, V), jnp.float32),
        ),
    )(x, router_W, router_b, emb0, woutr, bout)


def _q_body(emb_ref, wout_ref, t_ref, q_ref):
    eb = emb_ref[0]
    h = eb - eb[0:1, :]
    q_ref[0] = (jnp.dot(h, wout_ref[0], preferred_element_type=jnp.float32)
                + t_ref[...])


def _q_call(emb, Wout, t):
    return pl.pallas_call(
        _q_body,
        grid=(E,),
        in_specs=[
            pl.BlockSpec((1, V, D), lambda e: (e, 0, 0)),
            pl.BlockSpec((1, D, V), lambda e: (e, 0, 0)),
            pl.BlockSpec((1, V), lambda e: (0, 0)),
        ],
        out_specs=pl.BlockSpec((1, V, V), lambda e: (e, 0, 0)),
        out_shape=jax.ShapeDtypeStruct((E, V, V), jnp.float32),
    )(emb, Wout, t)


def _sc_body(q_hbm, idx0_hbm, idx1_hbm, w0_hbm, w1_hbm, out_hbm,
             idx0_v, idx1_v, w0_v, w1_v, buf0, buf1, outb, sem0, sem1):
    nc = 2
    b = lax.axis_index("s") * nc + lax.axis_index("c")
    pltpu.sync_copy(idx0_hbm.at[b], idx0_v)
    pltpu.sync_copy(idx1_hbm.at[b], idx1_v)
    pltpu.sync_copy(w0_hbm.at[b, pl.ds(0, LANES)], w0_v)
    pltpu.sync_copy(w1_hbm.at[b, pl.ds(0, LANES)], w1_v)
    w0 = w0_v[...]
    w1 = w1_v[...]

    def chunk_body(c, carry):
        cp0 = pltpu.async_copy(
            q_hbm.at[idx0_v.at[pl.ds(c * CH, CH)]], buf0, sem0)
        cp1 = pltpu.async_copy(
            q_hbm.at[idx1_v.at[pl.ds(c * CH, CH)]], buf1, sem1)
        cp0.wait()
        cp1.wait()

        def row_body(t, carry2):
            for j in range(V // LANES):
                sl = pl.ds(j * LANES, LANES)
                outb[t, sl] = buf0[t, sl] * w0 + buf1[t, sl] * w1
            return carry2

        lax.fori_loop(0, CH, row_body, 0, unroll=False)
        pltpu.sync_copy(outb, out_hbm.at[pl.ds(b * S + c * CH, CH)])
        return carry

    lax.fori_loop(0, S // CH, chunk_body, 0, unroll=False)


def _sc_call(qf, idx0, idx1, w0r, w1r):
    mesh = plsc.VectorSubcoreMesh(core_axis_name="c", subcore_axis_name="s")
    f = functools.partial(
        pl.kernel,
        mesh=mesh,
        out_type=jax.ShapeDtypeStruct((B * S, V), jnp.float32),
        scratch_types=[
            pltpu.VMEM((S,), jnp.int32),
            pltpu.VMEM((S,), jnp.int32),
            pltpu.VMEM((LANES,), jnp.float32),
            pltpu.VMEM((LANES,), jnp.float32),
            pltpu.VMEM((CH, V), jnp.float32),
            pltpu.VMEM((CH, V), jnp.float32),
            pltpu.VMEM((CH, V), jnp.float32),
            pltpu.SemaphoreType.DMA,
            pltpu.SemaphoreType.DMA,
        ],
    )(_sc_body)
    return f(qf, idx0, idx1, w0r, w1r)


def kernel(x, router_W, router_b, emb, Wout, bout):
    emb0 = emb[:, 0, :].reshape(1, E * D)
    woutr = Wout.reshape(E * D, V)
    idx0, idx1, w0r, w1r, t = _router_call(
        x, router_W, router_b.reshape(1, E), emb0, woutr, bout)
    q = _q_call(emb, Wout, t)
    out = _sc_call(q.reshape(E * V, V), idx0, idx1, w0r, w1r)
    return out.reshape(B, S, V)


# speedup vs baseline: 37.9057x; 1.1073x over previous
import functools

import jax
import jax.numpy as jnp
from jax import lax
from jax.experimental import pallas as pl
from jax.experimental.pallas import tpu as pltpu
from jax.experimental.pallas import tpu_sc as plsc

B, S, V, D, E = 32, 512, 512, 128, 8
CH = 32
NCHUNK = S // CH
LANES = 16


def _router_body(x_ref, rw_ref, rb_ref, emb0_ref, woutr_ref, bout_ref,
                 idx0_ref, idx1_ref, w0_ref, w1_ref, t_ref):
    xi = x_ref[...]
    xf = xi.astype(jnp.float32)
    logits = (jnp.dot(xf, rw_ref[...], preferred_element_type=jnp.float32)
              + rb_ref[...])
    iota = lax.broadcasted_iota(jnp.int32, (B, E), 1)
    m0 = jnp.max(logits, axis=1, keepdims=True)
    a0 = jnp.min(jnp.where(logits == m0, iota, E), axis=1)
    masked = jnp.where(iota == a0[:, None], -1e30, logits)
    m1 = jnp.max(masked, axis=1, keepdims=True)
    a1 = jnp.min(jnp.where(masked == m1, iota, E), axis=1)
    d = jnp.exp(m1[:, 0] - m0[:, 0])
    w0 = 1.0 / (1.0 + d)
    idx0_ref[...] = a0[:, None] * V + xi
    idx1_ref[...] = a1[:, None] * V + xi
    w0_ref[...] = jnp.broadcast_to(w0[:, None], (B, 128))
    w1_ref[...] = jnp.broadcast_to((1.0 - w0)[:, None], (B, 128))
    t_ref[...] = (jnp.dot(emb0_ref[...], woutr_ref[...],
                          preferred_element_type=jnp.float32)
                  + jnp.sum(bout_ref[...], axis=0, keepdims=True))


def _router_call(x, router_W, router_b, emb0, woutr, bout):
    return pl.pallas_call(
        _router_body,
        out_shape=(
            jax.ShapeDtypeStruct((B, S), jnp.int32),
            jax.ShapeDtypeStruct((B, S), jnp.int32),
            jax.ShapeDtypeStruct((B, 128), jnp.float32),
            jax.ShapeDtypeStruct((B, 128), jnp.float32),
            jax.ShapeDtypeStruct((1, V), jnp.float32),
        ),
    )(x, router_W, router_b, emb0, woutr, bout)


def _q_body(emb_ref, wout_ref, t_ref, q_ref):
    eb = emb_ref[0]
    h = eb - eb[0:1, :]
    q_ref[0] = (jnp.dot(h, wout_ref[0], preferred_element_type=jnp.float32)
                + t_ref[...])


def _q_call(emb, Wout, t):
    return pl.pallas_call(
        _q_body,
        grid=(E,),
        in_specs=[
            pl.BlockSpec((1, V, D), lambda e: (e, 0, 0)),
            pl.BlockSpec((1, D, V), lambda e: (e, 0, 0)),
            pl.BlockSpec((1, V), lambda e: (0, 0)),
        ],
        out_specs=pl.BlockSpec((1, V, V), lambda e: (e, 0, 0)),
        out_shape=jax.ShapeDtypeStruct((E, V, V), jnp.float32),
    )(emb, Wout, t)


def _sc_body(q_hbm, idx0_hbm, idx1_hbm, w0_hbm, w1_hbm, out_hbm,
             idx0_v, idx1_v, w0_v, w1_v, buf0, buf1, outb,
             g0a, g0b, g1a, g1b, wa, wb):
    nc = 2
    b = lax.axis_index("s") * nc + lax.axis_index("c")
    pltpu.sync_copy(idx0_hbm.at[b], idx0_v)
    pltpu.sync_copy(idx1_hbm.at[b], idx1_v)
    pltpu.sync_copy(w0_hbm.at[b, pl.ds(0, LANES)], w0_v)
    pltpu.sync_copy(w1_hbm.at[b, pl.ds(0, LANES)], w1_v)
    w0 = w0_v[...]
    w1 = w1_v[...]
    gsem = [(g0a, g1a), (g0b, g1b)]
    wsem = [wa, wb]

    def issue(c):
        s = c % 2
        cp0 = pltpu.async_copy(
            q_hbm.at[idx0_v.at[pl.ds(c * CH, CH)]], buf0.at[s], gsem[s][0])
        cp1 = pltpu.async_copy(
            q_hbm.at[idx1_v.at[pl.ds(c * CH, CH)]], buf1.at[s], gsem[s][1])
        return cp0, cp1

    pending = {0: issue(0)}
    writes = {}
    for c in range(NCHUNK):
        s = c % 2
        if c + 1 < NCHUNK:
            pending[c + 1] = issue(c + 1)
        cp0, cp1 = pending.pop(c)
        cp0.wait()
        cp1.wait()
        if c >= 2:
            writes.pop(c - 2).wait()

        def row_body(t, carry, s=s):
            for j in range(V // LANES):
                sl = pl.ds(j * LANES, LANES)
                outb[s, t, sl] = buf0[s, t, sl] * w0 + buf1[s, t, sl] * w1
            return carry

        lax.fori_loop(0, CH, row_body, 0, unroll=False)
        writes[c] = pltpu.async_copy(
            outb.at[s], out_hbm.at[pl.ds(b * S + c * CH, CH)], wsem[s])
    writes.pop(NCHUNK - 2).wait()
    writes.pop(NCHUNK - 1).wait()


def _sc_call(qf, idx0, idx1, w0r, w1r):
    mesh = plsc.VectorSubcoreMesh(core_axis_name="c", subcore_axis_name="s")
    f = functools.partial(
        pl.kernel,
        mesh=mesh,
        out_type=jax.ShapeDtypeStruct((B * S, V), jnp.float32),
        scratch_types=[
            pltpu.VMEM((S,), jnp.int32),
            pltpu.VMEM((S,), jnp.int32),
            pltpu.VMEM((LANES,), jnp.float32),
            pltpu.VMEM((LANES,), jnp.float32),
            pltpu.VMEM((2, CH, V), jnp.float32),
            pltpu.VMEM((2, CH, V), jnp.float32),
            pltpu.VMEM((2, CH, V), jnp.float32),
            pltpu.SemaphoreType.DMA,
            pltpu.SemaphoreType.DMA,
            pltpu.SemaphoreType.DMA,
            pltpu.SemaphoreType.DMA,
            pltpu.SemaphoreType.DMA,
            pltpu.SemaphoreType.DMA,
        ],
    )(_sc_body)
    return f(qf, idx0, idx1, w0r, w1r)


def kernel(x, router_W, router_b, emb, Wout, bout):
    emb0 = emb[:, 0, :].reshape(1, E * D)
    woutr = Wout.reshape(E * D, V)
    idx0, idx1, w0r, w1r, t = _router_call(
        x, router_W, router_b.reshape(1, E), emb0, woutr, bout)
    q = _q_call(emb, Wout, t)
    out = _sc_call(q.reshape(E * V, V), idx0, idx1, w0r, w1r)
    return out.reshape(B, S, V)
